# baseline (speedup 1.0000x reference)
import functools

import jax
import jax.numpy as jnp
from jax import lax
from jax.experimental import pallas as pl
from jax.experimental.pallas import tpu as pltpu
from jax.experimental.pallas import tpu_sc as plsc

N = 10000
E = 320000
D = 128
C = 20
EPS = 1e-5

NC = 2
NS = 16
NW = NC * NS
EPW = E // NW
K = 104
NCHUNK = 97
EPP = NCHUNK * K
M1 = 10016
RPT = 624
REM = N - NS * RPT


@functools.lru_cache(maxsize=None)
def _make_segsum():
  mesh = plsc.VectorSubcoreMesh(
      core_axis_name="c", subcore_axis_name="s",
      num_cores=NC, num_subcores=NS)

  @functools.partial(
      pl.kernel,
      out_type=jax.ShapeDtypeStruct((NC, N, D), jnp.float32),
      mesh=mesh,
      scratch_types=[
          pltpu.VMEM((EPP,), jnp.int32),
          pltpu.VMEM((NCHUNK, K), jnp.int32),
          pltpu.VMEM((K, D), jnp.float32),
          pltpu.VMEM((K, D), jnp.float32),
          pltpu.VMEM_SHARED((M1, D), jnp.float32),
          pltpu.SemaphoreType.DMA,
          pltpu.SemaphoreType.DMA,
      ],
  )
  def seg(src_hbm, dst_hbm, x_hbm, z_hbm, out_hbm,
          src_v, dst_v, buf_a, buf_b, acc, sem_a, sem_b):
    c = lax.axis_index("c")
    s = lax.axis_index("s")
    wid = s * NC + c
    pltpu.sync_copy(src_hbm.at[wid], src_v)
    pltpu.sync_copy(dst_hbm.at[wid], dst_v)
    pltpu.sync_copy(z_hbm.at[pl.ds(s * RPT, RPT)], acc.at[pl.ds(s * RPT, RPT)])

    @pl.when(s == NS - 1)
    def _():
      pltpu.sync_copy(z_hbm.at[pl.ds(NS * RPT, REM)], acc.at[pl.ds(NS * RPT, REM)])

    plsc.subcore_barrier()

    pltpu.async_copy(x_hbm.at[src_v.at[pl.ds(0, K)]], buf_a, sem_a)

    @pl.loop(0, (NCHUNK - 1) // 2)
    def _(i):
      j = i * 2
      pltpu.async_copy(x_hbm.at[src_v.at[pl.ds((j + 1) * K, K)]], buf_b, sem_b)
      pltpu.make_async_copy(x_hbm.at[src_v.at[pl.ds(j * K, K)]], buf_a, sem_a).wait()
      pltpu.sync_copy(buf_a, acc.at[dst_v.at[j]], add=True)
      pltpu.async_copy(x_hbm.at[src_v.at[pl.ds((j + 2) * K, K)]], buf_a, sem_a)
      pltpu.make_async_copy(x_hbm.at[src_v.at[pl.ds((j + 1) * K, K)]], buf_b, sem_b).wait()
      pltpu.sync_copy(buf_b, acc.at[dst_v.at[j + 1]], add=True)

    last = NCHUNK - 1
    pltpu.make_async_copy(x_hbm.at[src_v.at[pl.ds(last * K, K)]], buf_a, sem_a).wait()
    pltpu.sync_copy(buf_a, acc.at[dst_v.at[last]], add=True)

    plsc.subcore_barrier()
    pltpu.sync_copy(acc.at[pl.ds(s * RPT, RPT)],
                    out_hbm.at[c, pl.ds(s * RPT, RPT)])

    @pl.when(s == NS - 1)
    def _():
      pltpu.sync_copy(acc.at[pl.ds(NS * RPT, REM)],
                      out_hbm.at[c, pl.ds(NS * RPT, REM)])

  return seg


def _inorm(y):
  m = jnp.mean(y, axis=0, keepdims=True)
  v = jnp.mean((y - m) ** 2, axis=0, keepdims=True)
  return (y - m) * jax.lax.rsqrt(v + EPS)


def _stem_body(p_ref, w_ref, b_ref, x_ref, inv_ref):
  a = p_ref[0] + p_ref[1]
  inv = 1.0 / jnp.maximum(a[:, 4:5], 1.0)
  y = (jnp.dot(a[:, 0:16], w_ref[...], preferred_element_type=jnp.float32,
               precision=lax.Precision.HIGHEST)
       * inv + b_ref[...])
  x_ref[...] = jnp.maximum(_inorm(y), 0.0)
  inv_ref[...] = inv


def _mid_body(p_ref, inv_ref, w_ref, b_ref, h_ref):
  a = p_ref[0] + p_ref[1]
  y = (jnp.dot(a, w_ref[...], preferred_element_type=jnp.float32,
               precision=lax.Precision.HIGHEST)
       * inv_ref[...] + b_ref[...])
  h_ref[...] = jnp.maximum(_inorm(y), 0.0)


def _final_a_body(p_ref, inv_ref, w2_ref, b2_ref, y_ref):
  a = p_ref[0] + p_ref[1]
  y = (jnp.dot(a, w2_ref[...], preferred_element_type=jnp.float32,
               precision=lax.Precision.HIGHEST)
       * inv_ref[...] + b2_ref[...])
  y_ref[...] = _inorm(y)


def _final_b_body(y_ref, x_ref, wse_ref,
                  bse_ref, wlin_ref, blin_ref, lab_ref, wt_ref,
                  logits_ref, se_ref, loss_ref):
  x = x_ref[...]
  h2 = y_ref[...] + x
  xt = jnp.maximum(_inorm(h2), 0.0)
  se = _inorm(jnp.dot(xt, wse_ref[...], preferred_element_type=jnp.float32,
               precision=lax.Precision.HIGHEST)
              + bse_ref[...])
  se = jnp.where(se >= 0.0, se, 0.2 * se)
  x2 = se + xt
  logits = (jnp.dot(x2, wlin_ref[...], preferred_element_type=jnp.float32,
               precision=lax.Precision.HIGHEST)
            + blin_ref[...])
  m = jnp.max(logits, axis=1, keepdims=True)
  lse = jnp.log(jnp.sum(jnp.exp(logits - m), axis=1, keepdims=True)) + m
  lab = lab_ref[...]
  valid = (lab >= 0).astype(jnp.float32)
  labc = jnp.clip(lab, 0, C - 1)
  iota = lax.broadcasted_iota(jnp.int32, (N, C), 1)
  onehot = iota == labc
  sel = jnp.sum(jnp.where(onehot, logits, 0.0), axis=1, keepdims=True)
  w = jnp.sum(jnp.where(onehot, wt_ref[...], 0.0), axis=1, keepdims=True) * valid
  loss = jnp.sum((lse - sel) * w) / jnp.maximum(jnp.sum(w), EPS)
  logits_ref[...] = logits
  se_ref[...] = se
  loss_ref[...] = jnp.reshape(loss, (1, 1))


def kernel(coords, feat, label, weights, edge_index,
           W_in, b_in, W1, b1, W2, b2, W_se, b_se, W_lin, b_lin):
  f32 = jnp.float32
  pad = EPP - EPW
  src = jnp.pad(edge_index[0].astype(jnp.int32).reshape(NW, EPW),
                ((0, 0), (0, pad)))
  trash = (N + jnp.arange(NW, dtype=jnp.int32) // NC)[:, None]
  dst = jnp.concatenate(
      [edge_index[1].astype(jnp.int32).reshape(NW, EPW),
       jnp.broadcast_to(trash, (NW, pad))], axis=1).reshape(NW, NCHUNK, K)

  table_stem = jnp.concatenate(
      [feat.astype(f32), jnp.ones((N, 1), f32), jnp.zeros((N, D - 5), f32)],
      axis=1)
  z128 = jnp.zeros((N, D), f32)
  w16 = jnp.zeros((16, D), f32).at[0:4, :].set(W_in)
  segsum = _make_segsum()

  q = segsum(src, dst, table_stem, z128)
  x, inv = pl.pallas_call(
      _stem_body,
      out_shape=(jax.ShapeDtypeStruct((N, D), f32),
                 jax.ShapeDtypeStruct((N, 1), f32)),
  )(q, w16, b_in.reshape(1, D))

  p = segsum(src, dst, x, z128)
  h = pl.pallas_call(
      _mid_body,
      out_shape=jax.ShapeDtypeStruct((N, D), f32),
  )(p, inv, W1, b1.reshape(1, D))

  r = segsum(src, dst, h, z128)
  y2 = pl.pallas_call(
      _final_a_body,
      out_shape=jax.ShapeDtypeStruct((N, D), f32),
  )(r, inv, W2, b2.reshape(1, D))
  logits, se, loss = pl.pallas_call(
      _final_b_body,
      out_shape=(jax.ShapeDtypeStruct((N, C), f32),
                 jax.ShapeDtypeStruct((N, D), f32),
                 jax.ShapeDtypeStruct((1, 1), f32)),
  )(y2, x, W_se, b_se.reshape(1, D),
    W_lin, b_lin.reshape(1, C), label.astype(jnp.int32).reshape(N, 1),
    weights.reshape(1, C))

  return logits, se, loss.reshape(())


# speedup vs baseline: 1.5975x; 1.5975x over previous
import functools

import jax
import jax.numpy as jnp
from jax import lax
from jax.experimental import pallas as pl
from jax.experimental.pallas import tpu as pltpu
from jax.experimental.pallas import tpu_sc as plsc

N = 10000
E = 320000
D = 128
C = 20
EPS = 1e-5

NC = 2
NS = 16
NW = NC * NS
EPW = E // NW
K = 80
NCHUNK = EPW // K
EPP = EPW
M1 = N
RPT = 624
REM = N - NS * RPT


@functools.lru_cache(maxsize=None)
def _make_segsum():
  mesh = plsc.VectorSubcoreMesh(
      core_axis_name="c", subcore_axis_name="s",
      num_cores=NC, num_subcores=NS)

  @functools.partial(
      pl.kernel,
      out_type=jax.ShapeDtypeStruct((NC, N, D), jnp.float32),
      mesh=mesh,
      scratch_types=[
          pltpu.VMEM((EPP,), jnp.int32),
          pltpu.VMEM((NCHUNK, K), jnp.int32),
          pltpu.VMEM((K, D), jnp.float32),
          pltpu.VMEM((K, D), jnp.float32),
          pltpu.VMEM_SHARED((M1, D), jnp.float32),
          pltpu.SemaphoreType.DMA,
          pltpu.SemaphoreType.DMA,
      ],
  )
  def seg(src_hbm, dst_hbm, x_hbm, z_hbm, out_hbm,
          src_v, dst_v, buf_a, buf_b, acc, sem_a, sem_b):
    c = lax.axis_index("c")
    s = lax.axis_index("s")
    wid = s * NC + c
    pltpu.sync_copy(src_hbm.at[wid], src_v)
    pltpu.sync_copy(dst_hbm.at[wid], dst_v)
    pltpu.sync_copy(z_hbm.at[pl.ds(s * RPT, RPT)], acc.at[pl.ds(s * RPT, RPT)])

    @pl.when(s == NS - 1)
    def _():
      pltpu.sync_copy(z_hbm.at[pl.ds(NS * RPT, REM)], acc.at[pl.ds(NS * RPT, REM)])

    plsc.subcore_barrier()

    pltpu.async_copy(x_hbm.at[src_v.at[pl.ds(0, K)]], buf_a, sem_a)

    @pl.loop(0, (NCHUNK - 1) // 2)
    def _(i):
      j = i * 2
      pltpu.async_copy(x_hbm.at[src_v.at[pl.ds((j + 1) * K, K)]], buf_b, sem_b)
      pltpu.make_async_copy(x_hbm.at[src_v.at[pl.ds(j * K, K)]], buf_a, sem_a).wait()
      pltpu.sync_copy(buf_a, acc.at[dst_v.at[j]], add=True)
      pltpu.async_copy(x_hbm.at[src_v.at[pl.ds((j + 2) * K, K)]], buf_a, sem_a)
      pltpu.make_async_copy(x_hbm.at[src_v.at[pl.ds((j + 1) * K, K)]], buf_b, sem_b).wait()
      pltpu.sync_copy(buf_b, acc.at[dst_v.at[j + 1]], add=True)

    last = NCHUNK - 1
    pltpu.make_async_copy(x_hbm.at[src_v.at[pl.ds(last * K, K)]], buf_a, sem_a).wait()
    pltpu.sync_copy(buf_a, acc.at[dst_v.at[last]], add=True)

    plsc.subcore_barrier()
    pltpu.sync_copy(acc.at[pl.ds(s * RPT, RPT)],
                    out_hbm.at[c, pl.ds(s * RPT, RPT)])

    @pl.when(s == NS - 1)
    def _():
      pltpu.sync_copy(acc.at[pl.ds(NS * RPT, REM)],
                      out_hbm.at[c, pl.ds(NS * RPT, REM)])

  return seg


def _inorm(y):
  m = jnp.mean(y, axis=0, keepdims=True)
  v = jnp.mean((y - m) ** 2, axis=0, keepdims=True)
  return (y - m) * jax.lax.rsqrt(v + EPS)


def _stem_body(p_ref, w_ref, b_ref, x_ref, inv_ref):
  a = p_ref[0] + p_ref[1]
  inv = 1.0 / jnp.maximum(a[:, 4:5], 1.0)
  y = (jnp.dot(a[:, 0:16], w_ref[...], preferred_element_type=jnp.float32,
               precision=lax.Precision.HIGHEST)
       * inv + b_ref[...])
  x_ref[...] = jnp.maximum(_inorm(y), 0.0)
  inv_ref[...] = inv


def _mid_body(p_ref, inv_ref, w_ref, b_ref, h_ref):
  a = p_ref[0] + p_ref[1]
  y = (jnp.dot(a, w_ref[...], preferred_element_type=jnp.float32,
               precision=lax.Precision.HIGHEST)
       * inv_ref[...] + b_ref[...])
  h_ref[...] = jnp.maximum(_inorm(y), 0.0)


def _final_a_body(p_ref, inv_ref, w2_ref, b2_ref, y_ref):
  a = p_ref[0] + p_ref[1]
  y = (jnp.dot(a, w2_ref[...], preferred_element_type=jnp.float32,
               precision=lax.Precision.HIGHEST)
       * inv_ref[...] + b2_ref[...])
  y_ref[...] = _inorm(y)


def _final_b_body(y_ref, x_ref, wse_ref,
                  bse_ref, wlin_ref, blin_ref, lab_ref, wt_ref,
                  logits_ref, se_ref, loss_ref):
  x = x_ref[...]
  h2 = y_ref[...] + x
  xt = jnp.maximum(_inorm(h2), 0.0)
  se = _inorm(jnp.dot(xt, wse_ref[...], preferred_element_type=jnp.float32,
               precision=lax.Precision.HIGHEST)
              + bse_ref[...])
  se = jnp.where(se >= 0.0, se, 0.2 * se)
  x2 = se + xt
  logits = (jnp.dot(x2, wlin_ref[...], preferred_element_type=jnp.float32,
               precision=lax.Precision.HIGHEST)
            + blin_ref[...])
  m = jnp.max(logits, axis=1, keepdims=True)
  lse = jnp.log(jnp.sum(jnp.exp(logits - m), axis=1, keepdims=True)) + m
  lab = lab_ref[...]
  valid = (lab >= 0).astype(jnp.float32)
  labc = jnp.clip(lab, 0, C - 1)
  iota = lax.broadcasted_iota(jnp.int32, (N, C), 1)
  onehot = iota == labc
  sel = jnp.sum(jnp.where(onehot, logits, 0.0), axis=1, keepdims=True)
  w = jnp.sum(jnp.where(onehot, wt_ref[...], 0.0), axis=1, keepdims=True) * valid
  loss = jnp.sum((lse - sel) * w) / jnp.maximum(jnp.sum(w), EPS)
  logits_ref[...] = logits
  se_ref[...] = se
  loss_ref[...] = jnp.reshape(loss, (1, 1))


def kernel(coords, feat, label, weights, edge_index,
           W_in, b_in, W1, b1, W2, b2, W_se, b_se, W_lin, b_lin):
  f32 = jnp.float32
  src = edge_index[0].astype(jnp.int32).reshape(NW, EPW)
  dst = edge_index[1].astype(jnp.int32).reshape(NW, NCHUNK, K)

  table_stem = jnp.concatenate(
      [feat.astype(f32), jnp.ones((N, 1), f32), jnp.zeros((N, D - 5), f32)],
      axis=1)
  z128 = jnp.zeros((N, D), f32)
  w16 = jnp.zeros((16, D), f32).at[0:4, :].set(W_in)
  segsum = _make_segsum()

  q = segsum(src, dst, table_stem, z128)
  x, inv = pl.pallas_call(
      _stem_body,
      out_shape=(jax.ShapeDtypeStruct((N, D), f32),
                 jax.ShapeDtypeStruct((N, 1), f32)),
  )(q, w16, b_in.reshape(1, D))

  p = segsum(src, dst, x, z128)
  h = pl.pallas_call(
      _mid_body,
      out_shape=jax.ShapeDtypeStruct((N, D), f32),
  )(p, inv, W1, b1.reshape(1, D))

  r = segsum(src, dst, h, z128)
  y2 = pl.pallas_call(
      _final_a_body,
      out_shape=jax.ShapeDtypeStruct((N, D), f32),
  )(r, inv, W2, b2.reshape(1, D))
  logits, se, loss = pl.pallas_call(
      _final_b_body,
      out_shape=(jax.ShapeDtypeStruct((N, C), f32),
                 jax.ShapeDtypeStruct((N, D), f32),
                 jax.ShapeDtypeStruct((1, 1), f32)),
  )(y2, x, W_se, b_se.reshape(1, D),
    W_lin, b_lin.reshape(1, C), label.astype(jnp.int32).reshape(N, 1),
    weights.reshape(1, C))

  return logits, se, loss.reshape(())


# speedup vs baseline: 1.6459x; 1.0303x over previous
import functools

import jax
import jax.numpy as jnp
from jax import lax
from jax.experimental import pallas as pl
from jax.experimental.pallas import tpu as pltpu
from jax.experimental.pallas import tpu_sc as plsc

N = 10000
E = 320000
D = 128
C = 20
EPS = 1e-5

NC = 2
NS = 16
NW = NC * NS
EPW = E // NW
K = 80
NCHUNK = EPW // K
EPP = EPW
M1 = N
RPT = 624
REM = N - NS * RPT


@functools.lru_cache(maxsize=None)
def _make_segsum():
  mesh = plsc.VectorSubcoreMesh(
      core_axis_name="c", subcore_axis_name="s",
      num_cores=NC, num_subcores=NS)

  @functools.partial(
      pl.kernel,
      out_type=jax.ShapeDtypeStruct((NC, N, D), jnp.float32),
      mesh=mesh,
      scratch_types=[
          pltpu.VMEM((EPP,), jnp.int32),
          pltpu.VMEM((EPP,), jnp.int32),
          pltpu.VMEM((K, D), jnp.float32),
          pltpu.VMEM((K, D), jnp.float32),
          pltpu.VMEM_SHARED((M1, D), jnp.float32),
          pltpu.SemaphoreType.DMA,
          pltpu.SemaphoreType.DMA,
      ],
  )
  def seg(edge_hbm, x_hbm, z_hbm, out_hbm,
          src_v, dst_v, buf_a, buf_b, acc, sem_a, sem_b):
    c = lax.axis_index("c")
    s = lax.axis_index("s")
    wid = s * NC + c
    pltpu.sync_copy(edge_hbm.at[0, wid], src_v)
    pltpu.sync_copy(edge_hbm.at[1, wid], dst_v)
    pltpu.sync_copy(z_hbm.at[pl.ds(s * RPT, RPT)], acc.at[pl.ds(s * RPT, RPT)])

    @pl.when(s == NS - 1)
    def _():
      pltpu.sync_copy(z_hbm.at[pl.ds(NS * RPT, REM)], acc.at[pl.ds(NS * RPT, REM)])

    plsc.subcore_barrier()

    pltpu.async_copy(x_hbm.at[src_v.at[pl.ds(0, K)]], buf_a, sem_a)

    @pl.loop(0, (NCHUNK - 1) // 2)
    def _(i):
      j = i * 2
      pltpu.async_copy(x_hbm.at[src_v.at[pl.ds((j + 1) * K, K)]], buf_b, sem_b)
      pltpu.make_async_copy(x_hbm.at[src_v.at[pl.ds(j * K, K)]], buf_a, sem_a).wait()
      pltpu.sync_copy(buf_a, acc.at[dst_v.at[pl.ds(j * K, K)]], add=True)
      pltpu.async_copy(x_hbm.at[src_v.at[pl.ds((j + 2) * K, K)]], buf_a, sem_a)
      pltpu.make_async_copy(x_hbm.at[src_v.at[pl.ds((j + 1) * K, K)]], buf_b, sem_b).wait()
      pltpu.sync_copy(buf_b, acc.at[dst_v.at[pl.ds((j + 1) * K, K)]], add=True)

    last = NCHUNK - 1
    pltpu.make_async_copy(x_hbm.at[src_v.at[pl.ds(last * K, K)]], buf_a, sem_a).wait()
    pltpu.sync_copy(buf_a, acc.at[dst_v.at[pl.ds(last * K, K)]], add=True)

    plsc.subcore_barrier()
    pltpu.sync_copy(acc.at[pl.ds(s * RPT, RPT)],
                    out_hbm.at[c, pl.ds(s * RPT, RPT)])

    @pl.when(s == NS - 1)
    def _():
      pltpu.sync_copy(acc.at[pl.ds(NS * RPT, REM)],
                      out_hbm.at[c, pl.ds(NS * RPT, REM)])

  return seg


def _inorm(y):
  m = jnp.mean(y, axis=0, keepdims=True)
  v = jnp.mean((y - m) ** 2, axis=0, keepdims=True)
  return (y - m) * jax.lax.rsqrt(v + EPS)


def _stem_body(p_ref, w_ref, b_ref, x_ref, inv_ref):
  a = p_ref[0] + p_ref[1]
  inv = 1.0 / jnp.maximum(a[:, 4:5], 1.0)
  y = (jnp.dot(a[:, 0:16], w_ref[...], preferred_element_type=jnp.float32,
               precision=lax.Precision.HIGHEST)
       * inv + b_ref[...])
  x_ref[...] = jnp.maximum(_inorm(y), 0.0)
  inv_ref[...] = inv


def _mid_body(p_ref, inv_ref, w_ref, b_ref, h_ref):
  a = p_ref[0] + p_ref[1]
  y = (jnp.dot(a, w_ref[...], preferred_element_type=jnp.float32,
               precision=lax.Precision.HIGHEST)
       * inv_ref[...] + b_ref[...])
  h_ref[...] = jnp.maximum(_inorm(y), 0.0)


def _final_a_body(p_ref, inv_ref, w2_ref, b2_ref, y_ref):
  a = p_ref[0] + p_ref[1]
  y = (jnp.dot(a, w2_ref[...], preferred_element_type=jnp.float32,
               precision=lax.Precision.HIGHEST)
       * inv_ref[...] + b2_ref[...])
  y_ref[...] = _inorm(y)


def _final_b_body(y_ref, x_ref, wse_ref,
                  bse_ref, wlin_ref, blin_ref, wh_ref,
                  logits_ref, se_ref, loss_ref):
  x = x_ref[...]
  h2 = y_ref[...] + x
  xt = jnp.maximum(_inorm(h2), 0.0)
  se = _inorm(jnp.dot(xt, wse_ref[...], preferred_element_type=jnp.float32,
               precision=lax.Precision.HIGHEST)
              + bse_ref[...])
  se = jnp.where(se >= 0.0, se, 0.2 * se)
  x2 = se + xt
  logits = (jnp.dot(x2, wlin_ref[...], preferred_element_type=jnp.float32,
               precision=lax.Precision.HIGHEST)
            + blin_ref[...])
  m = jnp.max(logits, axis=1, keepdims=True)
  lse = jnp.log(jnp.sum(jnp.exp(logits - m), axis=1, keepdims=True)) + m
  wh = wh_ref[...]
  loss = jnp.sum(wh * (lse - logits)) / jnp.maximum(jnp.sum(wh), EPS)
  logits_ref[...] = logits
  se_ref[...] = se
  loss_ref[...] = jnp.reshape(loss, (1, 1))


def kernel(coords, feat, label, weights, edge_index,
           W_in, b_in, W1, b1, W2, b2, W_se, b_se, W_lin, b_lin):
  f32 = jnp.float32
  edges = edge_index.astype(jnp.int32).reshape(2, NW, EPW)
  wh = jnp.where((label[:, None] == jnp.arange(C, dtype=label.dtype)[None, :])
                 & (label >= 0)[:, None],
                 weights[None, :].astype(f32), 0.0)

  table_stem = jnp.concatenate(
      [feat.astype(f32), jnp.ones((N, 1), f32), jnp.zeros((N, D - 5), f32)],
      axis=1)
  z128 = jnp.zeros((N, D), f32)
  w16 = jnp.zeros((16, D), f32).at[0:4, :].set(W_in)
  segsum = _make_segsum()

  q = segsum(edges, table_stem, z128)
  x, inv = pl.pallas_call(
      _stem_body,
      out_shape=(jax.ShapeDtypeStruct((N, D), f32),
                 jax.ShapeDtypeStruct((N, 1), f32)),
  )(q, w16, b_in.reshape(1, D))

  p = segsum(edges, x, z128)
  h = pl.pallas_call(
      _mid_body,
      out_shape=jax.ShapeDtypeStruct((N, D), f32),
  )(p, inv, W1, b1.reshape(1, D))

  r = segsum(edges, h, z128)
  y2 = pl.pallas_call(
      _final_a_body,
      out_shape=jax.ShapeDtypeStruct((N, D), f32),
  )(r, inv, W2, b2.reshape(1, D))
  logits, se, loss = pl.pallas_call(
      _final_b_body,
      out_shape=(jax.ShapeDtypeStruct((N, C), f32),
                 jax.ShapeDtypeStruct((N, D), f32),
                 jax.ShapeDtypeStruct((1, 1), f32)),
  )(y2, x, W_se, b_se.reshape(1, D),
    W_lin, b_lin.reshape(1, C), wh)

  return logits, se, loss.reshape(())


# speedup vs baseline: 1.6678x; 1.0133x over previous
import functools

import jax
import jax.numpy as jnp
from jax import lax
from jax.experimental import pallas as pl
from jax.experimental.pallas import tpu as pltpu
from jax.experimental.pallas import tpu_sc as plsc

N = 10000
E = 320000
D = 128
C = 20
EPS = 1e-5

NC = 2
NS = 16
NW = NC * NS
EPW = E // NW
K = 80
NCHUNK = EPW // K
EPP = EPW
M1 = N
RPT = 624
REM = N - NS * RPT


@functools.lru_cache(maxsize=None)
def _make_segsum():
  mesh = plsc.VectorSubcoreMesh(
      core_axis_name="c", subcore_axis_name="s",
      num_cores=NC, num_subcores=NS)

  @functools.partial(
      pl.kernel,
      out_type=jax.ShapeDtypeStruct((NC, N, D), jnp.float32),
      mesh=mesh,
      scratch_types=[
          pltpu.VMEM((EPP,), jnp.int32),
          pltpu.VMEM((EPP,), jnp.int32),
          pltpu.VMEM((K, D), jnp.float32),
          pltpu.VMEM((K, D), jnp.float32),
          pltpu.VMEM_SHARED((M1, D), jnp.float32),
          pltpu.SemaphoreType.DMA,
          pltpu.SemaphoreType.DMA,
      ],
  )
  def seg(edge_hbm, x_hbm, z_hbm, out_hbm,
          src_v, dst_v, buf_a, buf_b, acc, sem_a, sem_b):
    c = lax.axis_index("c")
    s = lax.axis_index("s")
    wid = s * NC + c
    cp_s = pltpu.async_copy(edge_hbm.at[0, wid], src_v, sem_a)
    cp_d = pltpu.async_copy(edge_hbm.at[1, wid], dst_v, sem_b)
    pltpu.sync_copy(z_hbm.at[pl.ds(s * RPT, RPT)], acc.at[pl.ds(s * RPT, RPT)])
    cp_s.wait()
    cp_d.wait()

    @pl.when(s == NS - 1)
    def _():
      pltpu.sync_copy(z_hbm.at[pl.ds(NS * RPT, REM)], acc.at[pl.ds(NS * RPT, REM)])

    plsc.subcore_barrier()

    pltpu.async_copy(x_hbm.at[src_v.at[pl.ds(0, K)]], buf_a, sem_a)

    @pl.loop(0, (NCHUNK - 1) // 2)
    def _(i):
      j = i * 2
      pltpu.async_copy(x_hbm.at[src_v.at[pl.ds((j + 1) * K, K)]], buf_b, sem_b)
      pltpu.make_async_copy(x_hbm.at[src_v.at[pl.ds(j * K, K)]], buf_a, sem_a).wait()
      pltpu.sync_copy(buf_a, acc.at[dst_v.at[pl.ds(j * K, K)]], add=True)
      pltpu.async_copy(x_hbm.at[src_v.at[pl.ds((j + 2) * K, K)]], buf_a, sem_a)
      pltpu.make_async_copy(x_hbm.at[src_v.at[pl.ds((j + 1) * K, K)]], buf_b, sem_b).wait()
      pltpu.sync_copy(buf_b, acc.at[dst_v.at[pl.ds((j + 1) * K, K)]], add=True)

    last = NCHUNK - 1
    pltpu.make_async_copy(x_hbm.at[src_v.at[pl.ds(last * K, K)]], buf_a, sem_a).wait()
    pltpu.sync_copy(buf_a, acc.at[dst_v.at[pl.ds(last * K, K)]], add=True)

    plsc.subcore_barrier()
    pltpu.sync_copy(acc.at[pl.ds(s * RPT, RPT)],
                    out_hbm.at[c, pl.ds(s * RPT, RPT)])

    @pl.when(s == NS - 1)
    def _():
      pltpu.sync_copy(acc.at[pl.ds(NS * RPT, REM)],
                      out_hbm.at[c, pl.ds(NS * RPT, REM)])

  return seg


def _inorm(y):
  m = jnp.mean(y, axis=0, keepdims=True)
  v = jnp.mean((y - m) ** 2, axis=0, keepdims=True)
  return (y - m) * jax.lax.rsqrt(v + EPS)


def _stem_body(p_ref, w_ref, b_ref, x_ref, inv_ref):
  a = p_ref[0] + p_ref[1]
  inv = 1.0 / jnp.maximum(a[:, 4:5], 1.0)
  y = (jnp.dot(a[:, 0:16], w_ref[...], preferred_element_type=jnp.float32,
               precision=lax.Precision.HIGHEST)
       * inv + b_ref[...])
  x_ref[...] = jnp.maximum(_inorm(y), 0.0)
  inv_ref[...] = inv


def _mid_body(p_ref, inv_ref, w_ref, b_ref, h_ref):
  a = p_ref[0] + p_ref[1]
  y = (jnp.dot(a, w_ref[...], preferred_element_type=jnp.float32,
               precision=lax.Precision.HIGHEST)
       * inv_ref[...] + b_ref[...])
  h_ref[...] = jnp.maximum(_inorm(y), 0.0)


def _final_a_body(p_ref, inv_ref, w2_ref, b2_ref, y_ref):
  a = p_ref[0] + p_ref[1]
  y = (jnp.dot(a, w2_ref[...], preferred_element_type=jnp.float32,
               precision=lax.Precision.HIGHEST)
       * inv_ref[...] + b2_ref[...])
  y_ref[...] = _inorm(y)


def _final_b_body(y_ref, x_ref, wse_ref,
                  bse_ref, wlin_ref, blin_ref, wh_ref,
                  logits_ref, se_ref, loss_ref):
  x = x_ref[...]
  h2 = y_ref[...] + x
  xt = jnp.maximum(_inorm(h2), 0.0)
  se = _inorm(jnp.dot(xt, wse_ref[...], preferred_element_type=jnp.float32,
               precision=lax.Precision.HIGHEST)
              + bse_ref[...])
  se = jnp.where(se >= 0.0, se, 0.2 * se)
  x2 = se + xt
  logits = (jnp.dot(x2, wlin_ref[...], preferred_element_type=jnp.float32,
               precision=lax.Precision.HIGHEST)
            + blin_ref[...])
  m = jnp.max(logits, axis=1, keepdims=True)
  lse = jnp.log(jnp.sum(jnp.exp(logits - m), axis=1, keepdims=True)) + m
  wh = wh_ref[...]
  loss = jnp.sum(wh * (lse - logits)) / jnp.maximum(jnp.sum(wh), EPS)
  logits_ref[...] = logits
  se_ref[...] = se
  loss_ref[...] = jnp.reshape(loss, (1, 1))


def kernel(coords, feat, label, weights, edge_index,
           W_in, b_in, W1, b1, W2, b2, W_se, b_se, W_lin, b_lin):
  f32 = jnp.float32
  edges = edge_index.astype(jnp.int32).reshape(2, NW, EPW)
  wh = jnp.where((label[:, None] == jnp.arange(C, dtype=label.dtype)[None, :])
                 & (label >= 0)[:, None],
                 weights[None, :].astype(f32), 0.0)

  table_stem = jnp.concatenate(
      [feat.astype(f32), jnp.ones((N, 1), f32), jnp.zeros((N, D - 5), f32)],
      axis=1)
  z128 = jnp.zeros((N, D), f32)
  w16 = jnp.zeros((16, D), f32).at[0:4, :].set(W_in)
  segsum = _make_segsum()

  q = segsum(edges, table_stem, z128)
  x, inv = pl.pallas_call(
      _stem_body,
      out_shape=(jax.ShapeDtypeStruct((N, D), f32),
                 jax.ShapeDtypeStruct((N, 1), f32)),
  )(q, w16, b_in.reshape(1, D))

  p = segsum(edges, x, z128)
  h = pl.pallas_call(
      _mid_body,
      out_shape=jax.ShapeDtypeStruct((N, D), f32),
  )(p, inv, W1, b1.reshape(1, D))

  r = segsum(edges, h, z128)
  y2 = pl.pallas_call(
      _final_a_body,
      out_shape=jax.ShapeDtypeStruct((N, D), f32),
  )(r, inv, W2, b2.reshape(1, D))
  logits, se, loss = pl.pallas_call(
      _final_b_body,
      out_shape=(jax.ShapeDtypeStruct((N, C), f32),
                 jax.ShapeDtypeStruct((N, D), f32),
                 jax.ShapeDtypeStruct((1, 1), f32)),
  )(y2, x, W_se, b_se.reshape(1, D),
    W_lin, b_lin.reshape(1, C), wh)

  return logits, se, loss.reshape(())


# speedup vs baseline: 1.6888x; 1.0126x over previous
import functools

import jax
import jax.numpy as jnp
from jax import lax
from jax.experimental import pallas as pl
from jax.experimental.pallas import tpu as pltpu
from jax.experimental.pallas import tpu_sc as plsc

N = 10000
E = 320000
D = 128
C = 20
EPS = 1e-5

NC = 2
NS = 16
NW = NC * NS
EPW = E // NW
K = 80
NCHUNK = EPW // K
EPP = EPW
M1 = N
RPT = 624
REM = N - NS * RPT


@functools.lru_cache(maxsize=None)
def _make_segsum():
  mesh = plsc.VectorSubcoreMesh(
      core_axis_name="c", subcore_axis_name="s",
      num_cores=NC, num_subcores=NS)

  @functools.partial(
      pl.kernel,
      out_type=jax.ShapeDtypeStruct((NC, N, D), jnp.float32),
      mesh=mesh,
      scratch_types=[
          pltpu.VMEM((EPP,), jnp.int32),
          pltpu.VMEM((EPP,), jnp.int32),
          pltpu.VMEM((K, D), jnp.float32),
          pltpu.VMEM((K, D), jnp.float32),
          pltpu.VMEM_SHARED((M1, D), jnp.float32),
          pltpu.SemaphoreType.DMA,
          pltpu.SemaphoreType.DMA,
      ],
  )
  def seg(edge_hbm, x_hbm, z_hbm, out_hbm,
          src_v, dst_v, buf_a, buf_b, acc, sem_a, sem_b):
    c = lax.axis_index("c")
    s = lax.axis_index("s")
    wid = s * NC + c
    cp_s = pltpu.async_copy(edge_hbm.at[0, wid], src_v, sem_a)
    cp_d = pltpu.async_copy(edge_hbm.at[1, wid], dst_v, sem_b)
    pltpu.sync_copy(z_hbm.at[pl.ds(s * RPT, RPT)], acc.at[pl.ds(s * RPT, RPT)])
    cp_s.wait()
    cp_d.wait()

    @pl.when(s == NS - 1)
    def _():
      pltpu.sync_copy(z_hbm.at[pl.ds(NS * RPT, REM)], acc.at[pl.ds(NS * RPT, REM)])

    plsc.subcore_barrier()

    pltpu.async_copy(x_hbm.at[src_v.at[pl.ds(0, K)]], buf_a, sem_a)

    @pl.loop(0, (NCHUNK - 1) // 2)
    def _(i):
      j = i * 2
      pltpu.async_copy(x_hbm.at[src_v.at[pl.ds((j + 1) * K, K)]], buf_b, sem_b)
      pltpu.make_async_copy(x_hbm.at[src_v.at[pl.ds(j * K, K)]], buf_a, sem_a).wait()
      pltpu.sync_copy(buf_a, acc.at[dst_v.at[pl.ds(j * K, K)]], add=True)
      pltpu.async_copy(x_hbm.at[src_v.at[pl.ds((j + 2) * K, K)]], buf_a, sem_a)
      pltpu.make_async_copy(x_hbm.at[src_v.at[pl.ds((j + 1) * K, K)]], buf_b, sem_b).wait()
      pltpu.sync_copy(buf_b, acc.at[dst_v.at[pl.ds((j + 1) * K, K)]], add=True)

    last = NCHUNK - 1
    pltpu.make_async_copy(x_hbm.at[src_v.at[pl.ds(last * K, K)]], buf_a, sem_a).wait()
    pltpu.sync_copy(buf_a, acc.at[dst_v.at[pl.ds(last * K, K)]], add=True)

    plsc.subcore_barrier()
    pltpu.sync_copy(acc.at[pl.ds(s * RPT, RPT)],
                    out_hbm.at[c, pl.ds(s * RPT, RPT)])

    @pl.when(s == NS - 1)
    def _():
      pltpu.sync_copy(acc.at[pl.ds(NS * RPT, REM)],
                      out_hbm.at[c, pl.ds(NS * RPT, REM)])

  return seg


def _dot3(a, b):
  f32 = jnp.float32
  ah = a.astype(jnp.bfloat16)
  al = (a - ah.astype(f32)).astype(jnp.bfloat16)
  bh = b.astype(jnp.bfloat16)
  bl = (b - bh.astype(f32)).astype(jnp.bfloat16)
  d = lambda u, v: jnp.dot(u, v, preferred_element_type=f32)
  return d(ah, bh) + (d(al, bh) + d(ah, bl))


def _inorm(y):
  m = jnp.mean(y, axis=0, keepdims=True)
  v = jnp.mean((y - m) ** 2, axis=0, keepdims=True)
  return (y - m) * jax.lax.rsqrt(v + EPS)


def _stem_body(p_ref, w_ref, b_ref, x_ref, inv_ref):
  a = p_ref[0] + p_ref[1]
  inv = 1.0 / jnp.maximum(a[:, 4:5], 1.0)
  y = (jnp.dot(a[:, 0:16], w_ref[...], preferred_element_type=jnp.float32,
               precision=lax.Precision.HIGHEST)
       * inv + b_ref[...])
  x_ref[...] = jnp.maximum(_inorm(y), 0.0)
  inv_ref[...] = inv


def _mid_body(p_ref, inv_ref, w_ref, b_ref, h_ref):
  a = p_ref[0] + p_ref[1]
  y = (_dot3(a, w_ref[...])
       * inv_ref[...] + b_ref[...])
  h_ref[...] = jnp.maximum(_inorm(y), 0.0)


def _final_a_body(p_ref, inv_ref, w2_ref, b2_ref, y_ref):
  a = p_ref[0] + p_ref[1]
  y = (_dot3(a, w2_ref[...])
       * inv_ref[...] + b2_ref[...])
  y_ref[...] = _inorm(y)


def _final_b_body(y_ref, x_ref, wse_ref,
                  bse_ref, wlin_ref, blin_ref, wh_ref,
                  logits_ref, se_ref, loss_ref):
  x = x_ref[...]
  h2 = y_ref[...] + x
  xt = jnp.maximum(_inorm(h2), 0.0)
  se = _inorm(_dot3(xt, wse_ref[...])
              + bse_ref[...])
  se = jnp.where(se >= 0.0, se, 0.2 * se)
  x2 = se + xt
  logits = (_dot3(x2, wlin_ref[...])
            + blin_ref[...])
  m = jnp.max(logits, axis=1, keepdims=True)
  lse = jnp.log(jnp.sum(jnp.exp(logits - m), axis=1, keepdims=True)) + m
  wh = wh_ref[...]
  loss = jnp.sum(wh * (lse - logits)) / jnp.maximum(jnp.sum(wh), EPS)
  logits_ref[...] = logits
  se_ref[...] = se
  loss_ref[...] = jnp.reshape(loss, (1, 1))


def kernel(coords, feat, label, weights, edge_index,
           W_in, b_in, W1, b1, W2, b2, W_se, b_se, W_lin, b_lin):
  f32 = jnp.float32
  edges = edge_index.astype(jnp.int32).reshape(2, NW, EPW)
  wh = jnp.where((label[:, None] == jnp.arange(C, dtype=label.dtype)[None, :])
                 & (label >= 0)[:, None],
                 weights[None, :].astype(f32), 0.0)

  table_stem = jnp.concatenate(
      [feat.astype(f32), jnp.ones((N, 1), f32), jnp.zeros((N, D - 5), f32)],
      axis=1)
  z128 = jnp.zeros((N, D), f32)
  w16 = jnp.zeros((16, D), f32).at[0:4, :].set(W_in)
  segsum = _make_segsum()

  q = segsum(edges, table_stem, z128)
  x, inv = pl.pallas_call(
      _stem_body,
      out_shape=(jax.ShapeDtypeStruct((N, D), f32),
                 jax.ShapeDtypeStruct((N, 1), f32)),
  )(q, w16, b_in.reshape(1, D))

  p = segsum(edges, x, z128)
  h = pl.pallas_call(
      _mid_body,
      out_shape=jax.ShapeDtypeStruct((N, D), f32),
  )(p, inv, W1, b1.reshape(1, D))

  r = segsum(edges, h, z128)
  y2 = pl.pallas_call(
      _final_a_body,
      out_shape=jax.ShapeDtypeStruct((N, D), f32),
  )(r, inv, W2, b2.reshape(1, D))
  logits, se, loss = pl.pallas_call(
      _final_b_body,
      out_shape=(jax.ShapeDtypeStruct((N, C), f32),
                 jax.ShapeDtypeStruct((N, D), f32),
                 jax.ShapeDtypeStruct((1, 1), f32)),
  )(y2, x, W_se, b_se.reshape(1, D),
    W_lin, b_lin.reshape(1, C), wh)

  return logits, se, loss.reshape(())
